# baseline (speedup 1.0000x reference)
import jax
import jax.numpy as jnp
import numpy as np
from jax import lax
from jax.experimental import pallas as pl
from jax.experimental.pallas import tpu as pltpu
from jax.experimental.pallas import tpu_sc as plsc

B = 1024
NUM_NUM = 13
N_CAT = 26
CAT_VOCAB = 1000
D = 64
NC = 2
NS = 16
NW = NC * NS
RW = B // NW
CPW = RW * N_CAT
CW = 64
CCH = CPW // CW
VPC = CW // 16

_OFFS = (np.arange(CPW, dtype=np.int32) % N_CAT) * CAT_VOCAB


def _sc_body(catids_hbm, ctab_hbm, wlin_hbm, offs_hbm,
             s_hbm, q_hbm, wv_hbm,
             cidx, widx, crows, wvals, smv, qmv, sem):
    wid = lax.axis_index("s") * NC + lax.axis_index("c")
    base = wid * RW

    pltpu.sync_copy(catids_hbm.at[wid], cidx)
    pltpu.sync_copy(offs_hbm, widx)

    for c in range(CCH):
        for k in range(VPC):
            sl = pl.ds(k * 16, 16)
            t = cidx[c, sl] + widx[c, sl]
            cidx[c, sl] = t
            widx[c, sl] = t + NUM_NUM

    copies = []
    for c in range(CCH):
        copies.append(pltpu.async_copy(
            ctab_hbm.at[cidx.at[c]], crows.at[pl.ds(c * CW, CW)], sem))
    for c in range(CCH):
        copies.append(pltpu.async_copy(
            wlin_hbm.at[widx.at[c]], wvals.at[pl.ds(c * CW, CW)], sem))
    for cp in copies:
        cp.wait()

    pltpu.sync_copy(wvals, wv_hbm.at[pl.ds(wid * CPW, CPW)])

    zero = jnp.zeros((16,), jnp.float32)

    def row_body(r, _):
        s = [zero] * 4
        q = [zero] * 4
        cb = r * N_CAT
        for f in range(N_CAT):
            for c in range(4):
                v = crows[cb + f, pl.ds(c * 16, 16)]
                s[c] = s[c] + v
                q[c] = q[c] + v * v
        for c in range(4):
            smv[r, pl.ds(c * 16, 16)] = s[c]
            qmv[r, pl.ds(c * 16, 16)] = q[c]
        return 0

    lax.fori_loop(0, RW, row_body, 0)
    pltpu.sync_copy(smv, s_hbm.at[pl.ds(base, RW)])
    pltpu.sync_copy(qmv, q_hbm.at[pl.ds(base, RW)])


_sc_call = pl.kernel(
    _sc_body,
    out_type=(
        jax.ShapeDtypeStruct((B, D), jnp.float32),
        jax.ShapeDtypeStruct((B, D), jnp.float32),
        jax.ShapeDtypeStruct((NW * CPW,), jnp.float32),
    ),
    mesh=plsc.VectorSubcoreMesh(core_axis_name="c", subcore_axis_name="s"),
    scratch_types=[
        pltpu.VMEM((CCH, CW), jnp.int32),
        pltpu.VMEM((CCH, CW), jnp.int32),
        pltpu.VMEM((CPW, D), jnp.float32),
        pltpu.VMEM((CPW,), jnp.float32),
        pltpu.VMEM((RW, D), jnp.float32),
        pltpu.VMEM((RW, D), jnp.float32),
        pltpu.SemaphoreType.DMA,
    ],
    compiler_params=pltpu.CompilerParams(use_tc_tiling_on_sc=False),
)


def _tc_body(scat_ref, qcat_ref, wv_ref, nidx_ref, ntab_ref, wlin_ref,
             w1_ref, b1_ref, w2_ref, b2_ref, wout_ref, blin_ref, bout_ref,
             out_ref):
    nidx = nidx_ref[...]
    cols = [
        jnp.sum(jnp.where(nidx == i, 1.0, 0.0), axis=1, keepdims=True)
        for i in range(NUM_NUM)
    ]
    cnt = jnp.concatenate(cols, axis=1)
    ntab = ntab_ref[...]
    snum = jnp.dot(cnt, ntab, preferred_element_type=jnp.float32)
    qnum = jnp.dot(cnt, ntab * ntab, preferred_element_type=jnp.float32)
    s = scat_ref[...] + snum
    q = qcat_ref[...] + qnum
    fm = 0.5 * (s * s - q)

    x = jnp.dot(fm, w1_ref[...], preferred_element_type=jnp.float32)
    x = jnp.maximum(x + b1_ref[...], 0.0)
    x = jnp.dot(x, w2_ref[...], preferred_element_type=jnp.float32)
    x = jnp.maximum(x + b2_ref[...], 0.0)
    inter = jnp.dot(x, wout_ref[...], preferred_element_type=jnp.float32)
    catlin = jnp.sum(wv_ref[...], axis=1, keepdims=True)
    wnum = wlin_ref[0:NUM_NUM, :]
    numlin = jnp.dot(
        nidx.astype(jnp.float32), wnum, preferred_element_type=jnp.float32)
    out_ref[...] = inter + catlin + numlin + blin_ref[0, 0] + bout_ref[0, 0]


def kernel(numeric_inputs, categorical_inputs, W_linear, b_linear,
           numeric_table, cat_tables, W1, b1, W2, b2, Wout, bout):
    catids = categorical_inputs.reshape(NW, CCH, CW)
    offs = jnp.asarray(_OFFS.reshape(CCH, CW))
    cat_flat = cat_tables.reshape(N_CAT * CAT_VOCAB, D)
    wlin_flat = W_linear.reshape(NUM_NUM + N_CAT * CAT_VOCAB)

    scat, qcat, wv = _sc_call(catids, cat_flat, wlin_flat, offs)

    out = pl.pallas_call(
        _tc_body,
        out_shape=jax.ShapeDtypeStruct((B, 1), jnp.float32),
    )(
        scat,
        qcat,
        wv.reshape(B, N_CAT),
        numeric_inputs,
        numeric_table,
        W_linear,
        W1,
        b1.reshape(1, -1),
        W2,
        b2.reshape(1, -1),
        Wout,
        b_linear.reshape(1, 1),
        bout.reshape(1, 1),
    )
    return out


# speedup vs baseline: 1.1803x; 1.1803x over previous
import jax
import jax.numpy as jnp
from jax import lax
from jax.experimental import pallas as pl
from jax.experimental.pallas import tpu as pltpu
from jax.experimental.pallas import tpu_sc as plsc

B = 1024
NUM_NUM = 13
N_CAT = 26
CAT_VOCAB = 1000
D = 64
NC = 2
NS = 16
NW = NC * NS
RW = B // NW
CPW = RW * N_CAT
CCH = 8
CW = CPW // CCH


def _sc_body(catidx_hbm, ctab_hbm, wcat_hbm,
             s_hbm, q_hbm, wv_hbm,
             cidx, crows, wvals, smv, qmv, sem):
    wid = lax.axis_index("s") * NC + lax.axis_index("c")
    base = wid * RW

    pltpu.sync_copy(catidx_hbm.at[wid], cidx)

    copies = []
    for c in range(CCH):
        copies.append(pltpu.async_copy(
            ctab_hbm.at[cidx.at[c]], crows.at[pl.ds(c * CW, CW)], sem))
    for c in range(CCH):
        copies.append(pltpu.async_copy(
            wcat_hbm.at[cidx.at[c]], wvals.at[pl.ds(c * CW, CW)], sem))
    for cp in copies:
        cp.wait()

    pltpu.sync_copy(wvals, wv_hbm.at[pl.ds(wid * CPW, CPW)])

    zero = jnp.zeros((16,), jnp.float32)

    def row_body(r, _):
        s = [zero] * 4
        q = [zero] * 4
        cb = r * N_CAT
        for f in range(N_CAT):
            for c in range(4):
                v = crows[cb + f, pl.ds(c * 16, 16)]
                s[c] = s[c] + v
                q[c] = q[c] + v * v
        for c in range(4):
            smv[r, pl.ds(c * 16, 16)] = s[c]
            qmv[r, pl.ds(c * 16, 16)] = q[c]
        return 0

    lax.fori_loop(0, RW, row_body, 0)
    pltpu.sync_copy(smv, s_hbm.at[pl.ds(base, RW)])
    pltpu.sync_copy(qmv, q_hbm.at[pl.ds(base, RW)])


_sc_call = pl.kernel(
    _sc_body,
    out_type=(
        jax.ShapeDtypeStruct((B, D), jnp.float32),
        jax.ShapeDtypeStruct((B, D), jnp.float32),
        jax.ShapeDtypeStruct((NW * CPW,), jnp.float32),
    ),
    mesh=plsc.VectorSubcoreMesh(core_axis_name="c", subcore_axis_name="s"),
    scratch_types=[
        pltpu.VMEM((CCH, CW), jnp.int32),
        pltpu.VMEM((CPW, D), jnp.float32),
        pltpu.VMEM((CPW,), jnp.float32),
        pltpu.VMEM((RW, D), jnp.float32),
        pltpu.VMEM((RW, D), jnp.float32),
        pltpu.SemaphoreType.DMA,
    ],
    compiler_params=pltpu.CompilerParams(use_tc_tiling_on_sc=False),
)


def _tc_num_body(nidx_ref, ntab_ref, wnum_ref, snum_ref, qnum_ref, nlin_ref):
    nidx = nidx_ref[...]
    cols = [
        jnp.sum(jnp.where(nidx == i, 1.0, 0.0), axis=1, keepdims=True)
        for i in range(NUM_NUM)
    ]
    cnt = jnp.concatenate(cols, axis=1)
    ntab = ntab_ref[...]
    snum_ref[...] = jnp.dot(cnt, ntab, preferred_element_type=jnp.float32)
    qnum_ref[...] = jnp.dot(
        cnt, ntab * ntab, preferred_element_type=jnp.float32)
    nlin_ref[...] = jnp.sum(
        nidx.astype(jnp.float32) * wnum_ref[...], axis=1, keepdims=True)


def _tc_main_body(scat_ref, qcat_ref, snum_ref, qnum_ref, wv_ref, nlin_ref,
                  w1_ref, b1_ref, w2_ref, b2_ref, woutt_ref, bsum_ref,
                  out_ref):
    s = scat_ref[...] + snum_ref[...]
    q = qcat_ref[...] + qnum_ref[...]
    fm = 0.5 * (s * s - q)

    x = jnp.dot(fm, w1_ref[...], preferred_element_type=jnp.float32)
    x = jnp.maximum(x + b1_ref[...], 0.0)
    x = jnp.dot(x, w2_ref[...], preferred_element_type=jnp.float32)
    x = jnp.maximum(x + b2_ref[...], 0.0)
    inter = jnp.sum(x * woutt_ref[...], axis=1, keepdims=True)
    catlin = jnp.sum(wv_ref[...], axis=1, keepdims=True)
    out_ref[...] = inter + catlin + nlin_ref[...] + bsum_ref[0, 0]


def kernel(numeric_inputs, categorical_inputs, W_linear, b_linear,
           numeric_table, cat_tables, W1, b1, W2, b2, Wout, bout):
    cat_gidx = categorical_inputs + (
        jnp.arange(N_CAT, dtype=jnp.int32) * CAT_VOCAB)[None, :]
    cat_gidx = cat_gidx.reshape(NW, CCH, CW)

    cat_flat = cat_tables.reshape(N_CAT * CAT_VOCAB, D)
    wcat = W_linear[NUM_NUM:, 0]

    scat, qcat, wv = _sc_call(cat_gidx, cat_flat, wcat)

    snum, qnum, nlin = pl.pallas_call(
        _tc_num_body,
        out_shape=(
            jax.ShapeDtypeStruct((B, D), jnp.float32),
            jax.ShapeDtypeStruct((B, D), jnp.float32),
            jax.ShapeDtypeStruct((B, 1), jnp.float32),
        ),
    )(
        numeric_inputs,
        numeric_table,
        W_linear[:NUM_NUM, 0].reshape(1, NUM_NUM),
    )

    out = pl.pallas_call(
        _tc_main_body,
        out_shape=jax.ShapeDtypeStruct((B, 1), jnp.float32),
    )(
        scat,
        qcat,
        snum,
        qnum,
        wv.reshape(B, N_CAT),
        nlin,
        W1,
        b1.reshape(1, -1),
        W2,
        b2.reshape(1, -1),
        Wout.reshape(1, -1),
        (b_linear + bout).reshape(1, 1),
    )
    return out
